# baseline (speedup 1.0000x reference)
import functools

import jax
import jax.numpy as jnp
from jax import lax
from jax.experimental import pallas as pl
from jax.experimental.pallas import tpu as pltpu
from jax.experimental.pallas import tpu_sc as plsc

VOCAB = 1000000
EMBED = 64
BATCH = 16384
NEG = 10

NUM_CORES = 2
NUM_SUBCORES = 16
NUM_WORKERS = NUM_CORES * NUM_SUBCORES
ITEMS_PER_WORKER = BATCH // NUM_WORKERS
SUB = 64
NSUB = ITEMS_PER_WORKER // SUB
NCHUNK = EMBED // 16

POS_PART = BATCH * 16
TOTAL_PART = (BATCH + BATCH * NEG) * 16
PART_ROWS = TOTAL_PART // 128


def _sc_scores_kernel(widx_hbm, wlsb_hbm, cidx_hbm, clsb_hbm,
                      nidx_hbm, nlsb_hbm, wtab_hbm, ctab_hbm,
                      part_out,
                      widx, wlsb, cidx, clsb, nidx, nlsb,
                      xrows, yrows, nrows, ppart, npart, sem):
    wid = lax.axis_index("s") * NUM_CORES + lax.axis_index("c")
    base_w = wid * ITEMS_PER_WORKER

    pltpu.sync_copy(widx_hbm.at[pl.ds(base_w, ITEMS_PER_WORKER)], widx)
    pltpu.sync_copy(wlsb_hbm.at[pl.ds(base_w, ITEMS_PER_WORKER)],
                    wlsb.at[pl.ds(0, ITEMS_PER_WORKER)])
    pltpu.sync_copy(cidx_hbm.at[pl.ds(base_w, ITEMS_PER_WORKER)], cidx)
    pltpu.sync_copy(clsb_hbm.at[pl.ds(base_w, ITEMS_PER_WORKER)],
                    clsb.at[pl.ds(0, ITEMS_PER_WORKER)])
    for k in range(NEG):
        pltpu.sync_copy(nidx_hbm.at[pl.ds(k * BATCH + base_w, ITEMS_PER_WORKER)],
                        nidx.at[k])
        pltpu.sync_copy(nlsb_hbm.at[pl.ds(k * BATCH + base_w, ITEMS_PER_WORKER)],
                        nlsb.at[k, pl.ds(0, ITEMS_PER_WORKER)])

    for c in range(NSUB):
        lo = c * SUB
        cps = [pltpu.async_copy(wtab_hbm.at[widx.at[pl.ds(lo, SUB)]], xrows, sem),
               pltpu.async_copy(ctab_hbm.at[cidx.at[pl.ds(lo, SUB)]], yrows, sem)]
        for k in range(NEG):
            cps.append(pltpu.async_copy(
                ctab_hbm.at[nidx.at[k, pl.ds(lo, SUB)]], nrows.at[k], sem))
        for cp in cps:
            cp.wait()

        def item_body(i, _):
            g = lo + i
            hx = wlsb[pl.ds(g, 16)][0] * 64
            xs = [xrows[i, pl.ds(hx + j * 16, 16)] for j in range(NCHUNK)]
            hy = clsb[pl.ds(g, 16)][0] * 64
            acc = xs[0] * yrows[i, pl.ds(hy, 16)]
            for j in range(1, NCHUNK):
                acc = acc + xs[j] * yrows[i, pl.ds(hy + j * 16, 16)]
            ppart[pl.ds(i * 16, 16)] = acc
            for k in range(NEG):
                hn = nlsb[k, pl.ds(g, 16)][0] * 64
                acc = xs[0] * nrows[k, i, pl.ds(hn, 16)]
                for j in range(1, NCHUNK):
                    acc = acc + xs[j] * nrows[k, i, pl.ds(hn + j * 16, 16)]

                npart[pl.ds((i * NEG + k) * 16, 16)] = acc
            return 0

        lax.fori_loop(0, SUB, item_body, 0)

        base_c = base_w + lo
        pltpu.sync_copy(ppart, part_out.at[pl.ds(base_c * 16, SUB * 16)])
        pltpu.sync_copy(
            npart,
            part_out.at[pl.ds(POS_PART + base_c * NEG * 16, SUB * NEG * 16)])


def _loss_body(part_ref, out_ref):
    x = part_ref[...]
    r = lax.broadcasted_iota(jnp.int32, (128, 8), 0) // 16
    g = lax.broadcasted_iota(jnp.int32, (128, 8), 1)
    gmat = (r == g).astype(jnp.float32)
    s = jax.lax.dot_general(x, gmat, (((1,), (0,)), ((), ())),
                            preferred_element_type=jnp.float32)
    row = lax.broadcasted_iota(jnp.int32, (PART_ROWS, 8), 0)
    z = jnp.where(row < (BATCH * 16) // 128, s, -s)
    l = jnp.minimum(z, 0.0) - jnp.log1p(jnp.exp(-jnp.abs(z)))
    out_ref[...] = jnp.full((1, 1), -jnp.sum(l) / BATCH, jnp.float32)


def kernel(word, context, negative_samples, word_embed, ctx_embed):
    wtab = word_embed.reshape(VOCAB // 2, 2 * EMBED)
    ctab = ctx_embed.reshape(VOCAB // 2, 2 * EMBED)
    negt = negative_samples.T.reshape(-1)

    mesh = plsc.VectorSubcoreMesh(core_axis_name="c", subcore_axis_name="s")
    sc = functools.partial(
        pl.kernel,
        mesh=mesh,
        compiler_params=pltpu.CompilerParams(
            needs_layout_passes=False, use_tc_tiling_on_sc=True),
        out_type=jax.ShapeDtypeStruct((TOTAL_PART,), jnp.float32),
        scratch_types=[
            pltpu.VMEM((ITEMS_PER_WORKER,), jnp.int32),
            pltpu.VMEM((ITEMS_PER_WORKER + 16,), jnp.int32),
            pltpu.VMEM((ITEMS_PER_WORKER,), jnp.int32),
            pltpu.VMEM((ITEMS_PER_WORKER + 16,), jnp.int32),
            pltpu.VMEM((NEG, ITEMS_PER_WORKER), jnp.int32),
            pltpu.VMEM((NEG, ITEMS_PER_WORKER + 16), jnp.int32),
            pltpu.VMEM((SUB, 2 * EMBED), jnp.float32),
            pltpu.VMEM((SUB, 2 * EMBED), jnp.float32),
            pltpu.VMEM((NEG, SUB, 2 * EMBED), jnp.float32),
            pltpu.VMEM((SUB * 16,), jnp.float32),
            pltpu.VMEM((SUB * NEG * 16,), jnp.float32),
            pltpu.SemaphoreType.DMA,
        ],
    )(_sc_scores_kernel)
    part = sc(word >> 1, word & 1, context >> 1, context & 1,
              negt >> 1, negt & 1, wtab, ctab)

    loss2d = pl.pallas_call(
        _loss_body,
        out_shape=jax.ShapeDtypeStruct((1, 1), jnp.float32),
    )(part.reshape(PART_ROWS, 128))
    return loss2d[0, 0]


# speedup vs baseline: 1.5078x; 1.5078x over previous
import functools

import jax
import jax.numpy as jnp
from jax import lax
from jax.experimental import pallas as pl
from jax.experimental.pallas import tpu as pltpu
from jax.experimental.pallas import tpu_sc as plsc

VOCAB = 1000000
EMBED = 64
BATCH = 16384
NEG = 10

NUM_CORES = 2
NUM_SUBCORES = 16
NUM_WORKERS = NUM_CORES * NUM_SUBCORES
ITEMS_PER_WORKER = BATCH // NUM_WORKERS
SUB = 64
NSUB = ITEMS_PER_WORKER // SUB
NCHUNK = EMBED // 16

POS_PART = BATCH * 16
TOTAL_PART = (BATCH + BATCH * NEG) * 16
PART_ROWS = TOTAL_PART // 128


def _sc_scores_kernel(word_hbm, ctx_hbm, negs_hbm, wtab_hbm, ctab_hbm,
                      part_out,
                      widx, cidx, nidx, xrows, yrows, nrows,
                      ppart, npart, sem):
    wid = lax.axis_index("s") * NUM_CORES + lax.axis_index("c")
    base_w = wid * ITEMS_PER_WORKER

    pltpu.sync_copy(word_hbm.at[pl.ds(base_w, ITEMS_PER_WORKER)],
                    widx.at[pl.ds(0, ITEMS_PER_WORKER)])
    pltpu.sync_copy(ctx_hbm.at[pl.ds(base_w, ITEMS_PER_WORKER)],
                    cidx.at[pl.ds(0, ITEMS_PER_WORKER)])
    pltpu.sync_copy(negs_hbm.at[pl.ds(base_w * NEG, ITEMS_PER_WORKER * NEG)],
                    nidx.at[pl.ds(0, ITEMS_PER_WORKER * NEG)])

    for c in range(NSUB):
        lo = c * SUB

        def enq_body(i, _):
            g = lo + i
            rw = widx[pl.ds(g, 16)][0]
            pltpu.async_copy(wtab_hbm.at[rw], xrows.at[i], sem)
            rc = cidx[pl.ds(g, 16)][0]
            pltpu.async_copy(ctab_hbm.at[rc], yrows.at[i], sem)
            for k in range(NEG):
                rn = nidx[pl.ds(g * NEG + k, 16)][0]
                pltpu.async_copy(ctab_hbm.at[rn], nrows.at[i * NEG + k], sem)
            return 0

        lax.fori_loop(0, SUB, enq_body, 0)

        pltpu.make_async_copy(wtab_hbm.at[pl.ds(0, SUB), :], xrows, sem).wait()
        pltpu.make_async_copy(ctab_hbm.at[pl.ds(0, SUB), :], yrows, sem).wait()
        pltpu.make_async_copy(ctab_hbm.at[pl.ds(0, SUB * NEG), :], nrows,
                              sem).wait()

        def item_body(i, _):
            xs = [xrows[i, pl.ds(j * 16, 16)] for j in range(NCHUNK)]
            acc = xs[0] * yrows[i, pl.ds(0, 16)]
            for j in range(1, NCHUNK):
                acc = acc + xs[j] * yrows[i, pl.ds(j * 16, 16)]
            ppart[pl.ds(i * 16, 16)] = acc
            for k in range(NEG):
                acc = xs[0] * nrows[i * NEG + k, pl.ds(0, 16)]
                for j in range(1, NCHUNK):
                    acc = acc + xs[j] * nrows[i * NEG + k, pl.ds(j * 16, 16)]

                npart[pl.ds((i * NEG + k) * 16, 16)] = acc
            return 0

        lax.fori_loop(0, SUB, item_body, 0)

        base_c = base_w + lo
        pltpu.sync_copy(ppart, part_out.at[pl.ds(base_c * 16, SUB * 16)])
        pltpu.sync_copy(
            npart,
            part_out.at[pl.ds(POS_PART + base_c * NEG * 16, SUB * NEG * 16)])


def _loss_body(part_ref, out_ref):
    x = part_ref[...]
    r = lax.broadcasted_iota(jnp.int32, (128, 8), 0) // 16
    g = lax.broadcasted_iota(jnp.int32, (128, 8), 1)
    gmat = (r == g).astype(jnp.float32)
    s = jax.lax.dot_general(x, gmat, (((1,), (0,)), ((), ())),
                            preferred_element_type=jnp.float32)
    row = lax.broadcasted_iota(jnp.int32, (PART_ROWS, 8), 0)
    z = jnp.where(row < (BATCH * 16) // 128, s, -s)
    l = jnp.minimum(z, 0.0) - jnp.log1p(jnp.exp(-jnp.abs(z)))
    out_ref[...] = jnp.full((1, 1), -jnp.sum(l) / BATCH, jnp.float32)


def kernel(word, context, negative_samples, word_embed, ctx_embed):
    negs = negative_samples.reshape(-1)

    mesh = plsc.VectorSubcoreMesh(core_axis_name="c", subcore_axis_name="s")
    sc = functools.partial(
        pl.kernel,
        mesh=mesh,
        compiler_params=pltpu.CompilerParams(use_tc_tiling_on_sc=True),
        out_type=jax.ShapeDtypeStruct((TOTAL_PART,), jnp.float32),
        scratch_types=[
            pltpu.VMEM((ITEMS_PER_WORKER + 16,), jnp.int32),
            pltpu.VMEM((ITEMS_PER_WORKER + 16,), jnp.int32),
            pltpu.VMEM((ITEMS_PER_WORKER * NEG + 16,), jnp.int32),
            pltpu.VMEM((SUB, EMBED), jnp.float32),
            pltpu.VMEM((SUB, EMBED), jnp.float32),
            pltpu.VMEM((SUB * NEG, EMBED), jnp.float32),
            pltpu.VMEM((SUB * 16,), jnp.float32),
            pltpu.VMEM((SUB * NEG * 16,), jnp.float32),
            pltpu.SemaphoreType.DMA,
        ],
    )(_sc_scores_kernel)
    part = sc(word, context, negs, word_embed, ctx_embed)

    loss2d = pl.pallas_call(
        _loss_body,
        out_shape=jax.ShapeDtypeStruct((1, 1), jnp.float32),
    )(part.reshape(PART_ROWS, 128))
    return loss2d[0, 0]
